# Initial kernel scaffold; baseline (speedup 1.0000x reference)
import jax
import jax.numpy as jnp
from jax.experimental import pallas as pl


def kernel(x, adj, W1, b1, W2, b2):
    raise NotImplementedError("write your pallas kernel here")


# speedup vs baseline: 13.5543x; 13.5543x over previous
import functools

import jax
import jax.numpy as jnp
from jax import lax
from jax.experimental import pallas as pl
from jax.experimental.pallas import tpu as pltpu
from jax.experimental.pallas import tpu_sc as plsc

_N = 10000
_NP = 10240
_E = 320000
_NT = 32
_NSUB = 16
_CH = 128
_C = -(-_E // (_NT * _CH))
_EP = _NT * _C * _CH
_R = _NP // _NSUB
_ZR = 80
_BT = 1024

_mesh = plsc.VectorSubcoreMesh(core_axis_name="c", subcore_axis_name="s")


@functools.partial(
    pl.kernel,
    out_type=jax.ShapeDtypeStruct((_NT, _NP), jnp.float32),
    mesh=_mesh,
    compiler_params=pltpu.CompilerParams(needs_layout_passes=False),
    scratch_types=[
        pltpu.VMEM((_NP,), jnp.float32),
        pltpu.VMEM((_C, _CH), jnp.int32),
    ],
)
def _deg_kernel(dst_hbm, out_hbm, cnt, dstv):
    cid = lax.axis_index("c")
    sid = lax.axis_index("s")
    w = cid * _NSUB + sid
    pltpu.sync_copy(dst_hbm.at[w], dstv)

    @pl.loop(0, _NP // 16)
    def _(i):
        cnt[pl.ds(i * 16, 16)] = jnp.zeros((16,), jnp.float32)

    ones16 = jnp.ones((16,), jnp.float32)

    @pl.loop(0, _C)
    def _(ch):
        for j in range(_CH // 16):
            idx = dstv[ch, pl.ds(j * 16, 16)]
            plsc.addupdate_scatter(cnt, [idx], ones16)

    pltpu.sync_copy(cnt, out_hbm.at[w])


@functools.partial(
    pl.kernel,
    out_type=jax.ShapeDtypeStruct((2, _NP, 128), jnp.float32),
    mesh=_mesh,
    scratch_types=[
        pltpu.VMEM_SHARED((_NP, 128), jnp.float32),
        pltpu.VMEM((_C, _CH), jnp.int32),
        pltpu.VMEM((_C, _CH), jnp.int32),
        pltpu.VMEM((_CH, 128), jnp.float32),
        pltpu.VMEM((_ZR, 128), jnp.float32),
        pltpu.SemaphoreType.DMA,
    ],
)
def _scatter128(g_hbm, src_hbm, dst_hbm, out_hbm, acc, srcv, dstv, rows, stage, sem):
    cid = lax.axis_index("c")
    sid = lax.axis_index("s")
    w = cid * _NSUB + sid
    pltpu.sync_copy(src_hbm.at[w], srcv)
    pltpu.sync_copy(dst_hbm.at[w], dstv)

    @pl.loop(0, _ZR)
    def _(i):
        for j in range(8):
            stage[i, pl.ds(16 * j, 16)] = jnp.zeros((16,), jnp.float32)

    for k in range(_R // _ZR):
        pltpu.sync_copy(stage, acc.at[pl.ds(sid * _R + k * _ZR, _ZR)])
    plsc.subcore_barrier()

    @pl.loop(0, _C)
    def _(ch):
        pltpu.async_copy(g_hbm.at[srcv.at[ch]], rows, sem).wait()
        pltpu.sync_copy(rows, acc.at[dstv.at[ch]], add=True)

    plsc.subcore_barrier()
    for k in range(_R // _ZR):
        sl = pl.ds(sid * _R + k * _ZR, _ZR)
        pltpu.sync_copy(acc.at[sl], stage)
        pltpu.sync_copy(stage, out_hbm.at[cid, sl])


def _tc1_body(x_ref, w1_ref, dt_ref, g_ref, dis_ref):
    deg = jnp.sum(dt_ref[...], axis=1, keepdims=True) + 1.0
    dis = lax.rsqrt(deg)
    h = jnp.dot(x_ref[...], w1_ref[...], preferred_element_type=jnp.float32)
    g_ref[...] = h * dis
    dis_ref[...] = dis


def _tc2_body(p0, p1, g1, dis, b1, w2, o):
    z = jnp.maximum((p0[...] + p1[...] + g1[...]) * dis[...] + b1[...], 0.0)
    o[...] = jnp.dot(z, w2[...], preferred_element_type=jnp.float32) * dis[...]


def _tc3_body(q0, q1, g2, dis, b2, o):
    oo = (q0[...] + q1[...] + g2[...])[:, :64] * dis[...] + b2[...]
    m = jnp.max(oo, axis=1, keepdims=True)
    lse = m + jnp.log(jnp.sum(jnp.exp(oo - m), axis=1, keepdims=True))
    o[...] = oo - lse


def _row_spec(d):
    return pl.BlockSpec((_BT, d), lambda i: (i, 0))


def _full_spec(shape):
    return pl.BlockSpec(shape, lambda i: (0,) * len(shape))


def kernel(x, adj, W1, b1, W2, b2):
    pad = _EP - _E
    srcp = jnp.concatenate([adj[0], jnp.zeros((pad,), jnp.int32)]).reshape(_NT, _C, _CH)
    dstp = jnp.concatenate([adj[1], jnp.full((pad,), _N, jnp.int32)]).reshape(_NT, _C, _CH)
    xp = jnp.pad(x, ((0, _NP - _N), (0, 0)))

    dcnt_t = _deg_kernel(dstp).T

    g1, dis = pl.pallas_call(
        _tc1_body,
        grid=(_NP // _BT,),
        in_specs=[_row_spec(128), _full_spec((128, 128)), _row_spec(_NT)],
        out_specs=[_row_spec(128), _row_spec(1)],
        out_shape=[
            jax.ShapeDtypeStruct((_NP, 128), jnp.float32),
            jax.ShapeDtypeStruct((_NP, 1), jnp.float32),
        ],
    )(xp, W1, dcnt_t)

    p = _scatter128(g1, srcp, dstp)

    g2 = pl.pallas_call(
        _tc2_body,
        grid=(_NP // _BT,),
        in_specs=[_row_spec(128)] * 3 + [_row_spec(1)]
        + [_full_spec((1, 128)), _full_spec((128, 128))],
        out_specs=_row_spec(128),
        out_shape=jax.ShapeDtypeStruct((_NP, 128), jnp.float32),
    )(p[0], p[1], g1, dis, b1.reshape(1, 128), jnp.pad(W2, ((0, 0), (0, 64))))

    q = _scatter128(g2, srcp, dstp)

    o = pl.pallas_call(
        _tc3_body,
        grid=(_NP // _BT,),
        in_specs=[_row_spec(128)] * 3 + [_row_spec(1), _full_spec((1, 64))],
        out_specs=_row_spec(64),
        out_shape=jax.ShapeDtypeStruct((_NP, 64), jnp.float32),
    )(q[0], q[1], g2, dis, b2.reshape(1, 64))

    return o[:_N]
